# baseline (speedup 1.0000x reference)
import functools

import jax
import jax.numpy as jnp
from jax import lax
from jax.experimental import pallas as pl
from jax.experimental.pallas import tpu as pltpu
from jax.experimental.pallas import tpu_sc as plsc

B, L, V, D = 4096, 200, 1000000, 64

_info = plsc.get_sparse_core_info()
_NC, _NS, _LANES = _info.num_cores, _info.num_subcores, _info.num_lanes
NW = _NC * _NS
TOTAL = B * L
ROWS_W = TOTAL // NW
SUB = 128
CH = 512
NSUB = CH // SUB
NIT = ROWS_W // CH
NCOMBO = 3 * L


def _build():
    mesh = plsc.VectorSubcoreMesh(core_axis_name="c", subcore_axis_name="s")

    @functools.partial(
        pl.kernel,
        mesh=mesh,
        compiler_params=pltpu.CompilerParams(use_tc_tiling_on_sc=False,
                                             needs_layout_passes=False),
        out_type=jax.ShapeDtypeStruct((TOTAL, 2 * D), jnp.float32),
        scratch_types=[
            pltpu.VMEM((2, NSUB, SUB), jnp.int32),
            pltpu.VMEM((2, NSUB, SUB), jnp.int32),
            pltpu.VMEM((2, CH, D), jnp.float32),
            pltpu.VMEM((NCOMBO, D), jnp.float32),
            pltpu.SemaphoreType.DMA,
            pltpu.SemaphoreType.DMA,
            pltpu.SemaphoreType.DMA,
        ],
    )
    def emb_kernel(x2_hbm, c2_hbm, tok_hbm, combo_hbm, out_hbm,
                   xi_v, ci_v, tok_v, cmb_t, gsem, osem0, osem1):
        wid = lax.axis_index("s") * _NC + lax.axis_index("c")
        row0 = wid * (ROWS_W // SUB)
        osem = (osem0, osem1)
        iota = lax.iota(jnp.int32, _LANES)

        pltpu.sync_copy(combo_hbm, cmb_t)

        def idx_load(it, p):
            r = row0 + it * NSUB
            pltpu.sync_copy(x2_hbm.at[pl.ds(r, NSUB)], xi_v.at[p])
            pltpu.sync_copy(c2_hbm.at[pl.ds(r, NSUB)], ci_v.at[p])

        def gathers(p):
            for j in range(NSUB):
                dst = pl.ds(j * SUB, SUB)
                pltpu.async_copy(tok_hbm.at[xi_v.at[p, j]], tok_v.at[p, dst], gsem)

        def drain_g(p):
            dummy = out_hbm.at[pl.ds(0, SUB)]
            for j in range(NSUB):
                dst = pl.ds(j * SUB, SUB)
                pltpu.make_async_copy(dummy, tok_v.at[p, dst], gsem).wait()

        def out_issue(it, p):
            base = (row0 + it * NSUB) * SUB
            pltpu.async_copy(tok_v.at[p],
                             out_hbm.at[pl.ds(base, CH), pl.ds(0, D)], osem[p])

        def out_drain(it, p):
            base = (row0 + it * NSUB) * SUB
            pltpu.make_async_copy(tok_v.at[p],
                                  out_hbm.at[pl.ds(base, CH), pl.ds(0, D)],
                                  osem[p]).wait()

        def add(p):
            tp = tok_v.at[p]

            def body(g, rows):
                j = lax.shift_right_logical(g, 3)
                o = lax.shift_left(lax.bitwise_and(g, 7), 4)
                cidx16 = ci_v[p, j, pl.ds(o, _LANES)]
                for c4 in range(D):
                    colv = jnp.full((_LANES,), c4, jnp.int32)
                    tv = plsc.load_gather(tp, [rows, colv])
                    cv = plsc.load_gather(cmb_t, [cidx16, colv])
                    plsc.store_scatter(tp, [rows, colv], tv + cv)
                return rows + _LANES
            lax.fori_loop(0, CH // _LANES, body, iota, unroll=False)

        def step(it, p):
            idx_load(it + 1, 1 - p)
            out_drain(it - 1, 1 - p)
            gathers(1 - p)
            drain_g(p)
            add(p)
            out_issue(it, p)

        idx_load(0, 0)
        gathers(0)
        idx_load(1, 1)
        gathers(1)
        drain_g(0)
        add(0)
        out_issue(0, 0)

        def pair(k, c):
            step(2 * k + 1, 1)
            step(2 * k + 2, 0)
            return c

        lax.fori_loop(0, (NIT - 2) // 2, pair, 0)

        drain_g(1)
        add(1)
        out_issue(NIT - 1, 1)
        out_drain(NIT - 2, 0)
        out_drain(NIT - 1, 1)

    return emb_kernel


def kernel(x, seg_label, tok_table, seg_table, pos_embed):
    seq = x.shape[1]
    combo = (pos_embed[0, :seq, None, :] + seg_table[None, :, :]).reshape(3 * seq, D)
    cidx = seg_label.astype(jnp.int32) + 3 * jnp.arange(seq, dtype=jnp.int32)[None, :]
    x2 = x.astype(jnp.int32).reshape(TOTAL // SUB, SUB)
    c2 = cidx.reshape(TOTAL // SUB, SUB)
    out = _build()(x2, c2, tok_table, combo)
    return out.reshape(B, L, 2 * D)[:, :, :D]


# speedup vs baseline: 3.5531x; 3.5531x over previous
import functools

import jax
import jax.numpy as jnp
from jax import lax
from jax.experimental import pallas as pl
from jax.experimental.pallas import tpu as pltpu
from jax.experimental.pallas import tpu_sc as plsc

B, L, V, D = 4096, 200, 1000000, 64

_info = plsc.get_sparse_core_info()
_NC, _NS, _LANES = _info.num_cores, _info.num_subcores, _info.num_lanes
NW = _NC * _NS
TOTAL = B * L
ROWS_W = TOTAL // NW
SUB = 128
CH = 256
NSUB = CH // SUB
NIT = ROWS_W // CH
RU = 4


def _build():
    mesh = plsc.VectorSubcoreMesh(core_axis_name="c", subcore_axis_name="s")

    @functools.partial(
        pl.kernel,
        mesh=mesh,
        compiler_params=pltpu.CompilerParams(use_tc_tiling_on_sc=False),
        out_type=jax.ShapeDtypeStruct((TOTAL, 2 * D), jnp.float32),
        scratch_types=[
            pltpu.VMEM((2, NSUB, SUB), jnp.int32),
            pltpu.VMEM((2, NSUB, SUB), jnp.int32),
            pltpu.VMEM((2, CH, D), jnp.float32),
            pltpu.VMEM((2, CH, D), jnp.float32),
            pltpu.SemaphoreType.DMA,
            pltpu.SemaphoreType.DMA,
            pltpu.SemaphoreType.DMA,
        ],
    )
    def emb_kernel(x2_hbm, c2_hbm, tok_hbm, combo_hbm, out_hbm,
                   xi_v, ci_v, tok_v, cmb_v, gsem, osem0, osem1):
        wid = lax.axis_index("s") * _NC + lax.axis_index("c")
        row0 = wid * (ROWS_W // SUB)
        osem = (osem0, osem1)

        def idx_load(it, p):
            r = row0 + it * NSUB
            pltpu.sync_copy(x2_hbm.at[pl.ds(r, NSUB)], xi_v.at[p])
            pltpu.sync_copy(c2_hbm.at[pl.ds(r, NSUB)], ci_v.at[p])

        def gathers(p):
            for j in range(NSUB):
                dst = pl.ds(j * SUB, SUB)
                pltpu.async_copy(tok_hbm.at[xi_v.at[p, j]], tok_v.at[p, dst], gsem)
                pltpu.async_copy(combo_hbm.at[ci_v.at[p, j]], cmb_v.at[p, dst], gsem)

        def drain_g(p):
            dummy = out_hbm.at[pl.ds(0, SUB)]
            for j in range(NSUB):
                dst = pl.ds(j * SUB, SUB)
                pltpu.make_async_copy(dummy, tok_v.at[p, dst], gsem).wait()
                pltpu.make_async_copy(dummy, cmb_v.at[p, dst], gsem).wait()

        def out_issue(it, p):
            base = (row0 + it * NSUB) * SUB
            pltpu.async_copy(tok_v.at[p],
                             out_hbm.at[pl.ds(base, CH), pl.ds(0, D)], osem[p])

        def out_drain(it, p):
            base = (row0 + it * NSUB) * SUB
            pltpu.make_async_copy(tok_v.at[p],
                                  out_hbm.at[pl.ds(base, CH), pl.ds(0, D)],
                                  osem[p]).wait()

        def add(p):
            def body(i, c):
                for rr in range(RU):
                    r = i * RU + rr
                    for c4 in range(D // _LANES):
                        sl = pl.ds(c4 * _LANES, _LANES)
                        tok_v[p, r, sl] = tok_v[p, r, sl] + cmb_v[p, r, sl]
                return c
            lax.fori_loop(0, CH // RU, body, 0, unroll=False)

        def step(it, p):
            idx_load(it + 1, 1 - p)
            out_drain(it - 1, 1 - p)
            gathers(1 - p)
            drain_g(p)
            add(p)
            out_issue(it, p)

        idx_load(0, 0)
        gathers(0)
        idx_load(1, 1)
        gathers(1)
        drain_g(0)
        add(0)
        out_issue(0, 0)

        def pair(k, c):
            step(2 * k + 1, 1)
            step(2 * k + 2, 0)
            return c

        lax.fori_loop(0, (NIT - 2) // 2, pair, 0)

        drain_g(1)
        add(1)
        out_issue(NIT - 1, 1)
        out_drain(NIT - 2, 0)
        out_drain(NIT - 1, 1)

    return emb_kernel


def kernel(x, seg_label, tok_table, seg_table, pos_embed):
    seq = x.shape[1]
    combo = (pos_embed[0, :seq, None, :] + seg_table[None, :, :]).reshape(3 * seq, D)
    combo = jnp.tile(combo, (NW, 1))
    cidx = seg_label.astype(jnp.int32) + 3 * jnp.arange(seq, dtype=jnp.int32)[None, :]
    crep = cidx.reshape(TOTAL) + (3 * seq) * (jnp.arange(TOTAL, dtype=jnp.int32) // ROWS_W)
    x2 = x.astype(jnp.int32).reshape(TOTAL // SUB, SUB)
    c2 = crep.reshape(TOTAL // SUB, SUB)
    out = _build()(x2, c2, tok_table, combo)
    return out.reshape(B, L, 2 * D)[:, :, :D]


# speedup vs baseline: 3.5654x; 1.0035x over previous
import functools

import jax
import jax.numpy as jnp
from jax import lax
from jax.experimental import pallas as pl
from jax.experimental.pallas import tpu as pltpu
from jax.experimental.pallas import tpu_sc as plsc

B, L, V, D = 4096, 200, 1000000, 64

_info = plsc.get_sparse_core_info()
_NC, _NS, _LANES = _info.num_cores, _info.num_subcores, _info.num_lanes
NW = _NC * _NS
TOTAL = B * L
ROWS_W = TOTAL // NW
SUB = 128
CH = 256
NSUB = CH // SUB
NIT = ROWS_W // CH
RU = 4


def _build():
    mesh = plsc.VectorSubcoreMesh(core_axis_name="c", subcore_axis_name="s")

    @functools.partial(
        pl.kernel,
        mesh=mesh,
        compiler_params=pltpu.CompilerParams(use_tc_tiling_on_sc=False),
        out_type=jax.ShapeDtypeStruct((TOTAL, 2 * D), jnp.float32),
        scratch_types=[
            pltpu.VMEM((2, NSUB, 2, SUB), jnp.int32),
            pltpu.VMEM((2, CH, D), jnp.float32),
            pltpu.VMEM((2, CH, D), jnp.float32),
            pltpu.SemaphoreType.DMA,
            pltpu.SemaphoreType.DMA,
            pltpu.SemaphoreType.DMA,
        ],
    )
    def emb_kernel(i2_hbm, tok_hbm, combo_hbm, out_hbm,
                   i_v, tok_v, cmb_v, gsem, osem0, osem1):
        wid = lax.axis_index("s") * _NC + lax.axis_index("c")
        row0 = wid * (ROWS_W // SUB)
        osem = (osem0, osem1)

        def idx_load(it, p):
            r = row0 + it * NSUB
            pltpu.sync_copy(i2_hbm.at[pl.ds(r, NSUB)], i_v.at[p])

        def gathers(p):
            for j in range(NSUB):
                dst = pl.ds(j * SUB, SUB)
                pltpu.async_copy(tok_hbm.at[i_v.at[p, j, 0]], tok_v.at[p, dst], gsem)
                pltpu.async_copy(combo_hbm.at[i_v.at[p, j, 1]], cmb_v.at[p, dst], gsem)

        def drain_g(p):
            dummy = out_hbm.at[pl.ds(0, SUB)]
            for j in range(NSUB):
                dst = pl.ds(j * SUB, SUB)
                pltpu.make_async_copy(dummy, tok_v.at[p, dst], gsem).wait()
                pltpu.make_async_copy(dummy, cmb_v.at[p, dst], gsem).wait()

        def out_issue(it, p):
            base = (row0 + it * NSUB) * SUB
            pltpu.async_copy(tok_v.at[p],
                             out_hbm.at[pl.ds(base, CH), pl.ds(0, D)], osem[p])

        def out_drain(it, p):
            base = (row0 + it * NSUB) * SUB
            pltpu.make_async_copy(tok_v.at[p],
                                  out_hbm.at[pl.ds(base, CH), pl.ds(0, D)],
                                  osem[p]).wait()

        def add(p):
            def body(i, c):
                for rr in range(RU):
                    r = i * RU + rr
                    for c4 in range(D // _LANES):
                        sl = pl.ds(c4 * _LANES, _LANES)
                        tok_v[p, r, sl] = tok_v[p, r, sl] + cmb_v[p, r, sl]
                return c
            lax.fori_loop(0, CH // RU, body, 0, unroll=False)

        def step(it, p):
            idx_load(it + 1, 1 - p)
            out_drain(it - 1, 1 - p)
            gathers(1 - p)
            drain_g(p)
            add(p)
            out_issue(it, p)

        idx_load(0, 0)
        gathers(0)
        idx_load(1, 1)
        gathers(1)
        drain_g(0)
        add(0)
        out_issue(0, 0)

        def pair(k, c):
            step(2 * k + 1, 1)
            step(2 * k + 2, 0)
            return c

        lax.fori_loop(0, (NIT - 2) // 2, pair, 0)

        drain_g(1)
        add(1)
        out_issue(NIT - 1, 1)
        out_drain(NIT - 2, 0)
        out_drain(NIT - 1, 1)

    return emb_kernel


def kernel(x, seg_label, tok_table, seg_table, pos_embed):
    seq = x.shape[1]
    combo = (pos_embed[0, :seq, None, :] + seg_table[None, :, :]).reshape(3 * seq, D)
    combo = jnp.tile(combo, (64, 1))
    cidx = seg_label.astype(jnp.int32) + 3 * jnp.arange(seq, dtype=jnp.int32)[None, :]
    crep = cidx.reshape(TOTAL) + (3 * seq) * (jnp.arange(TOTAL, dtype=jnp.int32) & 63)
    i2 = jnp.stack([x.astype(jnp.int32).reshape(TOTAL // SUB, SUB),
                    crep.reshape(TOTAL // SUB, SUB)], axis=1)
    out = _build()(i2, tok_table, combo)
    return out.reshape(B, L, 2 * D)[:, :, :D]
